# baseline (speedup 1.0000x reference)
import functools

import jax
import jax.numpy as jnp
from jax import lax
from jax.experimental import pallas as pl
from jax.experimental.pallas import tpu as pltpu
from jax.experimental.pallas import tpu_sc as plsc

B = 16384
NW = 16
RPW = B // NW
NCHUNK = RPW // 16

_C = (
    3.3800903853631326e-08,
    0.9999942754839866,
    -0.4998385997133066,
    0.3315488284671695,
    -0.2398267798252278,
    0.16582375894772883,
    -0.09325294514616135,
    0.034850128855032095,
    -0.006151545067004348,
)

_DNUMS = lax.GatherDimensionNumbers(
    offset_dims=(), collapsed_slice_dims=(0,), start_index_map=(0,)
)


def _vgather(v, idx):
    return lax.gather(v, idx[:, None], _DNUMS, (1,),
                      mode=lax.GatherScatterMode.PROMISE_IN_BOUNDS)


_mesh = plsc.VectorSubcoreMesh(
    core_axis_name="c", subcore_axis_name="s", num_cores=1
)


@functools.partial(
    pl.kernel,
    out_type=(
        jax.ShapeDtypeStruct((1,), jnp.float32),
        jax.ShapeDtypeStruct((B,), jnp.int32),
    ),
    mesh=_mesh,
    scratch_types=[
        pltpu.VMEM((2 * RPW,), jnp.float32),
        pltpu.VMEM((RPW,), jnp.int32),
        pltpu.VMEM((RPW,), jnp.int32),
        pltpu.VMEM((16,), jnp.float32),
        pltpu.VMEM((16 * NW,), jnp.float32),
        pltpu.VMEM_SHARED((16 * NW,), jnp.float32),
    ],
)
def _sc_classifier(ctx_hbm, lab_hbm, loss_hbm, out_hbm,
                   ctx_v, lab_v, out_v, part_v, all_v, shared):
    wid = lax.axis_index("s")
    base = wid * RPW
    pltpu.sync_copy(ctx_hbm.at[pl.ds(2 * base, 2 * RPW)], ctx_v)
    pltpu.sync_copy(lab_hbm.at[pl.ds(base, RPW)], lab_v)

    iota = lax.iota(jnp.int32, 16)
    c_even = (iota * 2) & 15
    c_odd = (iota * 2 + 1) & 15
    lo_half = iota < 8

    acc = jnp.zeros((16,), jnp.float32)
    for j in range(NCHUNK):
        va = ctx_v[pl.ds(j * 32, 16)]
        vb = ctx_v[pl.ds(j * 32 + 16, 16)]
        lab = lab_v[pl.ds(j * 16, 16)]
        x0 = jnp.where(lo_half, _vgather(va, c_even), _vgather(vb, c_even))
        x1 = jnp.where(lo_half, _vgather(va, c_odd), _vgather(vb, c_odd))
        t = jnp.abs(x0 - x1)
        m = jnp.maximum(x0, x1)
        u = jnp.exp(-t)
        p = jnp.float32(_C[8])
        for c in _C[7::-1]:
            p = p * u + jnp.float32(c)
        sel = jnp.where(lab != 0, x1, x0)
        acc = acc + (m + p - sel)
        out_v[pl.ds(j * 16, 16)] = jnp.where(x1 > x0, 1, 0).astype(jnp.int32)

    pltpu.sync_copy(out_v, out_hbm.at[pl.ds(base, RPW)])
    part_v[...] = acc
    pltpu.sync_copy(part_v, shared.at[pl.ds(wid * 16, 16)])
    plsc.subcore_barrier()

    @pl.when(wid == 0)
    def _():
        pltpu.sync_copy(shared, all_v)
        tot = all_v[pl.ds(0, 16)]
        for i in range(1, NW):
            tot = tot + all_v[pl.ds(i * 16, 16)]
        for s in (8, 4, 2, 1):
            tot = tot + _vgather(tot, iota ^ s)
        part_v[...] = tot * jnp.float32(1.0 / B)
        pltpu.sync_copy(part_v.at[pl.ds(0, 1)], loss_hbm)


def kernel(soc_bag_embedding, context_bag_embedding, label):
    del soc_bag_embedding
    ctx_flat = context_bag_embedding.reshape(-1)
    loss_vec, out = _sc_classifier(ctx_flat, label)
    return loss_vec.reshape(()), out


# speedup vs baseline: 1.3842x; 1.3842x over previous
import functools

import jax
import jax.numpy as jnp
from jax import lax
from jax.experimental import pallas as pl
from jax.experimental.pallas import tpu as pltpu
from jax.experimental.pallas import tpu_sc as plsc

B = 16384
NW = 16
RPW = B // NW
NCHUNK = RPW // 16

_C = (
    3.3800903853631326e-08,
    0.9999942754839866,
    -0.4998385997133066,
    0.3315488284671695,
    -0.2398267798252278,
    0.16582375894772883,
    -0.09325294514616135,
    0.034850128855032095,
    -0.006151545067004348,
)

_DNUMS = lax.GatherDimensionNumbers(
    offset_dims=(), collapsed_slice_dims=(0,), start_index_map=(0,)
)


def _vgather(v, idx):
    return lax.gather(v, idx[:, None], _DNUMS, (1,),
                      mode=lax.GatherScatterMode.PROMISE_IN_BOUNDS)


_mesh = plsc.VectorSubcoreMesh(
    core_axis_name="c", subcore_axis_name="s", num_cores=1
)


@functools.partial(
    pl.kernel,
    out_type=(
        jax.ShapeDtypeStruct((1,), jnp.float32),
        jax.ShapeDtypeStruct((B,), jnp.int32),
    ),
    mesh=_mesh,
    scratch_types=[
        pltpu.VMEM((RPW,), jnp.float32),
        pltpu.VMEM((RPW,), jnp.float32),
        pltpu.VMEM((RPW,), jnp.int32),
        pltpu.VMEM((RPW,), jnp.int32),
        pltpu.VMEM((16,), jnp.float32),
        pltpu.VMEM((16 * NW,), jnp.float32),
        pltpu.VMEM_SHARED((16 * NW,), jnp.float32),
    ],
)
def _sc_classifier(x0_hbm, x1_hbm, lab_hbm, loss_hbm, out_hbm,
                   x0_v, x1_v, lab_v, out_v, part_v, all_v, shared):
    wid = lax.axis_index("s")
    base = wid * RPW
    pltpu.sync_copy(x0_hbm.at[pl.ds(base, RPW)], x0_v)
    pltpu.sync_copy(x1_hbm.at[pl.ds(base, RPW)], x1_v)
    pltpu.sync_copy(lab_hbm.at[pl.ds(base, RPW)], lab_v)

    iota = lax.iota(jnp.int32, 16)

    acc = jnp.zeros((16,), jnp.float32)
    for j in range(NCHUNK):
        x0 = x0_v[pl.ds(j * 16, 16)]
        x1 = x1_v[pl.ds(j * 16, 16)]
        lab = lab_v[pl.ds(j * 16, 16)]
        t = jnp.abs(x0 - x1)
        m = jnp.maximum(x0, x1)
        u = jnp.exp(-t)
        p = jnp.float32(_C[8])
        for c in _C[7::-1]:
            p = p * u + jnp.float32(c)
        sel = jnp.where(lab != 0, x1, x0)
        acc = acc + (m + p - sel)
        out_v[pl.ds(j * 16, 16)] = jnp.where(x1 > x0, 1, 0).astype(jnp.int32)

    pltpu.sync_copy(out_v, out_hbm.at[pl.ds(base, RPW)])
    part_v[...] = acc
    pltpu.sync_copy(part_v, shared.at[pl.ds(wid * 16, 16)])
    plsc.subcore_barrier()

    @pl.when(wid == 0)
    def _():
        pltpu.sync_copy(shared, all_v)
        tot = all_v[pl.ds(0, 16)]
        for i in range(1, NW):
            tot = tot + all_v[pl.ds(i * 16, 16)]
        for s in (8, 4, 2, 1):
            tot = tot + _vgather(tot, iota ^ s)
        part_v[...] = tot * jnp.float32(1.0 / B)
        pltpu.sync_copy(part_v.at[pl.ds(0, 1)], loss_hbm)


def kernel(soc_bag_embedding, context_bag_embedding, label):
    del soc_bag_embedding
    x0 = context_bag_embedding[:, 0]
    x1 = context_bag_embedding[:, 1]
    loss_vec, out = _sc_classifier(x0, x1, label)
    return loss_vec.reshape(()), out


# speedup vs baseline: 1.4663x; 1.0593x over previous
import functools

import jax
import jax.numpy as jnp
from jax import lax
from jax.experimental import pallas as pl
from jax.experimental.pallas import tpu as pltpu
from jax.experimental.pallas import tpu_sc as plsc

B = 16384
NW = 16
RPW = B // NW
NBLK = RPW // 128

_C = (
    9.962257064788371e-06,
    0.9992357023013462,
    -0.4902317394112386,
    0.2852745292150585,
    -0.13158319907983593,
    0.030449331759638636,
)

_DNUMS = lax.GatherDimensionNumbers(
    offset_dims=(), collapsed_slice_dims=(0,), start_index_map=(0,)
)


def _vgather(v, idx):
    return lax.gather(v, idx[:, None], _DNUMS, (1,),
                      mode=lax.GatherScatterMode.PROMISE_IN_BOUNDS)


_mesh = plsc.VectorSubcoreMesh(
    core_axis_name="c", subcore_axis_name="s", num_cores=1
)


@functools.partial(
    pl.kernel,
    out_type=(
        jax.ShapeDtypeStruct((1,), jnp.float32),
        jax.ShapeDtypeStruct((B,), jnp.int32),
    ),
    mesh=_mesh,
    scratch_types=[
        pltpu.VMEM((2 * RPW,), jnp.float32),
        pltpu.VMEM((RPW,), jnp.int32),
        pltpu.VMEM((RPW,), jnp.int32),
        pltpu.VMEM((16,), jnp.float32),
        pltpu.VMEM((16 * NW,), jnp.float32),
        pltpu.VMEM_SHARED((16 * NW,), jnp.float32),
        pltpu.SemaphoreType.DMA,
        pltpu.SemaphoreType.DMA,
        pltpu.SemaphoreType.DMA,
    ],
)
def _sc_classifier(y_hbm, lab_hbm, loss_hbm, out_hbm,
                   y_v, lab_v, out_v, part_v, all_v, shared,
                   sem_y, sem_lab, sem_out):
    wid = lax.axis_index("s")
    base = wid * RPW
    cy = pltpu.async_copy(y_hbm.at[pl.ds(2 * base, 2 * RPW)], y_v, sem_y)
    cl = pltpu.async_copy(lab_hbm.at[pl.ds(base, RPW)], lab_v, sem_lab)
    cy.wait()
    cl.wait()

    iota = lax.iota(jnp.int32, 16)

    accs = [jnp.zeros((16,), jnp.float32) for _ in range(4)]
    for k in range(NBLK):
        for i in range(8):
            off0 = k * 256 + i * 16
            r = k * 128 + i * 16
            x0 = y_v[pl.ds(off0, 16)]
            x1 = y_v[pl.ds(off0 + 128, 16)]
            lab = lab_v[pl.ds(r, 16)]
            t = jnp.abs(x0 - x1)
            m = jnp.maximum(x0, x1)
            u = jnp.exp(-t)
            p = jnp.float32(_C[5])
            for c in _C[4::-1]:
                p = p * u + jnp.float32(c)
            sel = jnp.where(lab != 0, x1, x0)
            accs[i % 4] = accs[i % 4] + (m + p - sel)
            out_v[pl.ds(r, 16)] = jnp.where(x1 > x0, 1, 0).astype(jnp.int32)

    co = pltpu.async_copy(out_v, out_hbm.at[pl.ds(base, RPW)], sem_out)
    part_v[...] = (accs[0] + accs[1]) + (accs[2] + accs[3])
    pltpu.sync_copy(part_v, shared.at[pl.ds(wid * 16, 16)])
    plsc.subcore_barrier()

    @pl.when(wid == 0)
    def _():
        pltpu.sync_copy(shared, all_v)
        tot = all_v[pl.ds(0, 16)]
        for i in range(1, NW):
            tot = tot + all_v[pl.ds(i * 16, 16)]
        for s in (8, 4, 2, 1):
            tot = tot + _vgather(tot, iota ^ s)
        part_v[...] = tot * jnp.float32(1.0 / B)
        pltpu.sync_copy(part_v.at[pl.ds(0, 1)], loss_hbm)

    co.wait()


def kernel(soc_bag_embedding, context_bag_embedding, label):
    del soc_bag_embedding
    y = jnp.swapaxes(context_bag_embedding.reshape(128, 128, 2), 1, 2).reshape(-1)
    loss_vec, out = _sc_classifier(y, label)
    return loss_vec.reshape(()), out


# speedup vs baseline: 1.5204x; 1.0369x over previous
import functools

import jax
import jax.numpy as jnp
from jax import lax
from jax.experimental import pallas as pl
from jax.experimental.pallas import tpu as pltpu
from jax.experimental.pallas import tpu_sc as plsc

B = 16384
NW = 16
RPW = B // NW
NBLK = RPW // 128

_C = (
    9.962257064788371e-06,
    0.9992357023013462,
    -0.4902317394112386,
    0.2852745292150585,
    -0.13158319907983593,
    0.030449331759638636,
)

_DNUMS = lax.GatherDimensionNumbers(
    offset_dims=(), collapsed_slice_dims=(0,), start_index_map=(0,)
)


def _vgather(v, idx):
    return lax.gather(v, idx[:, None], _DNUMS, (1,),
                      mode=lax.GatherScatterMode.PROMISE_IN_BOUNDS)


_mesh = plsc.VectorSubcoreMesh(
    core_axis_name="c", subcore_axis_name="s", num_cores=1
)


@functools.partial(
    pl.kernel,
    out_type=(
        jax.ShapeDtypeStruct((1,), jnp.float32),
        jax.ShapeDtypeStruct((B,), jnp.int32),
    ),
    mesh=_mesh,
    scratch_types=[
        pltpu.VMEM((2 * RPW,), jnp.float32),
        pltpu.VMEM((RPW,), jnp.int32),
        pltpu.VMEM((RPW,), jnp.int32),
        pltpu.VMEM((16,), jnp.float32),
        pltpu.VMEM((16 * NW,), jnp.float32),
        pltpu.VMEM_SHARED((16 * NW,), jnp.float32),
        pltpu.SemaphoreType.DMA,
        pltpu.SemaphoreType.DMA,
        pltpu.SemaphoreType.DMA,
    ],
)
def _sc_classifier(y_hbm, lab_hbm, loss_hbm, out_hbm,
                   y_v, lab_v, out_v, part_v, all_v, shared,
                   sem_y, sem_lab, sem_out):
    wid = lax.axis_index("s")
    base = wid * RPW
    cy = pltpu.async_copy(y_hbm.at[pl.ds(2 * base, 2 * RPW)], y_v, sem_y)
    cl = pltpu.async_copy(lab_hbm.at[pl.ds(base, RPW)], lab_v, sem_lab)
    cy.wait()
    cl.wait()

    iota = lax.iota(jnp.int32, 16)

    def block(k, accs):
        accs = list(accs)
        for i in range(8):
            off0 = k * 256 + i * 16
            r = k * 128 + i * 16
            x0 = y_v[pl.ds(off0, 16)]
            x1 = y_v[pl.ds(off0 + 128, 16)]
            lab = lab_v[pl.ds(r, 16)]
            t = jnp.abs(x0 - x1)
            m = jnp.maximum(x0, x1)
            u = jnp.exp(-t)
            p = jnp.float32(_C[5])
            for c in _C[4::-1]:
                p = p * u + jnp.float32(c)
            sel = jnp.where(lab != 0, x1, x0)
            accs[i % 4] = accs[i % 4] + (m + p - sel)
            out_v[pl.ds(r, 16)] = jnp.where(x1 > x0, 1, 0).astype(jnp.int32)
        return tuple(accs)

    zero = jnp.zeros((16,), jnp.float32)
    accs = lax.fori_loop(0, NBLK, block, (zero, zero, zero, zero))

    co = pltpu.async_copy(out_v, out_hbm.at[pl.ds(base, RPW)], sem_out)
    part_v[...] = (accs[0] + accs[1]) + (accs[2] + accs[3])
    pltpu.sync_copy(part_v, shared.at[pl.ds(wid * 16, 16)])
    plsc.subcore_barrier()

    @pl.when(wid == 0)
    def _():
        pltpu.sync_copy(shared, all_v)
        tot = all_v[pl.ds(0, 16)]
        for i in range(1, NW):
            tot = tot + all_v[pl.ds(i * 16, 16)]
        for s in (8, 4, 2, 1):
            tot = tot + _vgather(tot, iota ^ s)
        part_v[...] = tot * jnp.float32(1.0 / B)
        pltpu.sync_copy(part_v.at[pl.ds(0, 1)], loss_hbm)

    co.wait()


def kernel(soc_bag_embedding, context_bag_embedding, label):
    del soc_bag_embedding
    y = jnp.swapaxes(context_bag_embedding.reshape(128, 128, 2), 1, 2).reshape(-1)
    loss_vec, out = _sc_classifier(y, label)
    return loss_vec.reshape(()), out
